# baseline (speedup 1.0000x reference)
import functools

import jax
import jax.numpy as jnp
from jax.experimental import pallas as pl
from jax.experimental.pallas import tpu as pltpu
from jax.experimental.pallas import tpu_sc as plsc

N = 4096
D = 1024
E = 8
K = 2
H = 4 * D

B_R = 256
P = N * K + E * B_R
NBLK = P // B_R

TILE = 128
NT = N // TILE

NC, NS = 2, 16
NW = NC * NS
TOK_W = N // NW
CH = 32
CH2 = 16

NCHK = 16
CAP = 4
CW1 = D // (NCHK // 2)
CW2 = H // (NCHK // 2)

_f32 = jnp.float32
_i32 = jnp.int32


def _router_body(x_ref, grad_ref, wx_ref, wg_ref, rb_ref,
                 probs_ref, w0_ref, w1_ref, pos0_ref, pos1_ref, bexp_ref,
                 pfe_ref, plo_ref, phi_ref, flo_ref, fhi_ref, par_ref,
                 oh0_s, oh1_s, part0_s, part1_s):
    x = x_ref[...]
    logits = jnp.dot(x, wx_ref[...], preferred_element_type=_f32)
    logits = logits + grad_ref[...] * wg_ref[...] + rb_ref[...]

    m = jnp.max(logits, axis=1, keepdims=True)
    ex = jnp.exp(logits - m)
    probs = ex / jnp.sum(ex, axis=1, keepdims=True)
    probs_ref[...] = probs

    iota_e = jax.lax.broadcasted_iota(_i32, (N, E), 1)
    v0 = jnp.max(probs, axis=1, keepdims=True)
    e0 = jnp.min(jnp.where(probs == v0, iota_e, E), axis=1, keepdims=True)
    oh0 = (iota_e == e0).astype(_f32)
    pm = jnp.where(iota_e == e0, -1.0, probs)
    v1 = jnp.max(pm, axis=1, keepdims=True)
    e1 = jnp.min(jnp.where(pm == v1, iota_e, E), axis=1, keepdims=True)
    oh1 = (iota_e == e1).astype(_f32)

    w0_ref[...] = jnp.broadcast_to(v0, (N, 16))
    w1_ref[...] = jnp.broadcast_to(v1, (N, 16))
    oh0_s[...] = oh0
    oh1_s[...] = oh1

    ti = jax.lax.broadcasted_iota(_i32, (TILE, TILE), 0)
    tj = jax.lax.broadcasted_iota(_i32, (TILE, TILE), 1)
    tri = (tj < ti).astype(_f32)

    def tile_body(b, carry):
        run0, run1 = carry
        o0 = oh0_s[pl.ds(b * TILE, TILE), :]
        o1 = oh1_s[pl.ds(b * TILE, TILE), :]
        ex0 = jnp.dot(tri, o0, preferred_element_type=_f32)
        ex1 = jnp.dot(tri, o1, preferred_element_type=_f32)
        part0_s[pl.ds(b * TILE, TILE), :] = jnp.sum(o0 * (ex0 + run0), axis=1,
                                                    keepdims=True)
        part1_s[pl.ds(b * TILE, TILE), :] = jnp.sum(o1 * (ex1 + run1), axis=1,
                                                    keepdims=True)
        return (run0 + jnp.sum(o0, axis=0, keepdims=True),
                run1 + jnp.sum(o1, axis=0, keepdims=True))

    run0, run1 = jax.lax.fori_loop(
        0, NT, tile_body,
        (jnp.zeros((1, E), _f32), jnp.zeros((1, E), _f32)))

    tot = (run0 + run1).astype(_i32)
    padded = ((tot + (B_R - 1)) // B_R) * B_R

    offs = [jnp.zeros((1, 1), _i32)]
    acc = jnp.zeros((1, 1), _i32)
    for j in range(1, E):
        acc = acc + padded[:, j - 1:j]
        offs.append(acc)
    offi = jnp.concatenate(offs, axis=1)
    off = offi.astype(_f32)
    ends = off + padded.astype(_f32)

    pos0 = part0_s[...] + jnp.sum(oh0 * off, axis=1, keepdims=True)
    pos1 = part1_s[...] + jnp.sum(oh1 * (off + run0), axis=1, keepdims=True)
    pos0_ref[...] = pos0.astype(_i32)
    pos1_ref[...] = pos1.astype(_i32)

    rstart = (jax.lax.broadcasted_iota(_i32, (NBLK, E), 0) * B_R).astype(_f32)
    cnt = jnp.sum((rstart >= ends).astype(_i32), axis=1, keepdims=True)
    bexp_v = jnp.minimum(cnt, E - 1)
    bexp_ref[...] = bexp_v

    present = (padded > 0).astype(_i32)
    rs = offi // B_R
    runlen = padded // B_R

    ne_l, lp_l, ord_l = [], [], []
    for e in range(E):
        nv = jnp.full((1, 1), E, _i32)
        for j in range(E - 1, e, -1):
            nv = jnp.where(present[:, j:j + 1] == 1, j, nv)
        ne_l.append(nv)
        pv = jnp.full((1, 1), -1, _i32)
        lp = jnp.zeros((1, 1), _i32)
        for j in range(0, e):
            pv = jnp.where(present[:, j:j + 1] == 1, j, pv)
        for j in range(0, e):
            lp = jnp.where(pv == j, runlen[:, j:j + 1], lp)
        lp_l.append(lp)
        ov = jnp.zeros((1, 1), _i32)
        for j in range(0, e):
            ov = ov + present[:, j:j + 1]
        ord_l.append(ov)
    ne = jnp.concatenate(ne_l, axis=1)
    lenprev = jnp.concatenate(lp_l, axis=1)
    ordv = jnp.concatenate(ord_l, axis=1)

    iota_be = jax.lax.broadcasted_iota(_i32, (NBLK, E), 1)
    ohb = (iota_be == bexp_v).astype(_i32)

    def sel(v):
        return jnp.sum(ohb * v, axis=1, keepdims=True)

    rs_b = sel(rs)
    pfe_b = sel(ne)
    lp_b = sel(lenprev)
    par_b = sel(ordv) % 2
    present_b = sel(present)
    rblk = jax.lax.broadcasted_iota(_i32, (NBLK, 1), 0)
    i_b = rblk - rs_b
    has_next = (pfe_b < E).astype(_i32)
    pfe_ref[...] = jnp.minimum(pfe_b, E - 1)
    plo_ref[...] = jnp.minimum(NCHK, i_b * CAP) * has_next
    phi_ref[...] = jnp.minimum(NCHK, (i_b + 1) * CAP) * has_next
    runstart = (i_b == 0).astype(_i32) * present_b
    done_prev = jnp.minimum(NCHK, lp_b * CAP)
    flo_ref[...] = jnp.where(runstart == 1, done_prev, 0)
    fhi_ref[...] = jnp.where(runstart == 1, NCHK, 0)
    par_ref[...] = par_b


def _router(x, grad, wx, wgrow, rb):
    return pl.pallas_call(
        _router_body,
        out_shape=(
            jax.ShapeDtypeStruct((N, E), _f32),
            jax.ShapeDtypeStruct((N, 16), _f32),
            jax.ShapeDtypeStruct((N, 16), _f32),
            jax.ShapeDtypeStruct((N, 1), _i32),
            jax.ShapeDtypeStruct((N, 1), _i32),
            jax.ShapeDtypeStruct((NBLK, 1), _i32),
            jax.ShapeDtypeStruct((NBLK, 1), _i32),
            jax.ShapeDtypeStruct((NBLK, 1), _i32),
            jax.ShapeDtypeStruct((NBLK, 1), _i32),
            jax.ShapeDtypeStruct((NBLK, 1), _i32),
            jax.ShapeDtypeStruct((NBLK, 1), _i32),
            jax.ShapeDtypeStruct((NBLK, 1), _i32),
        ),
        scratch_shapes=[
            pltpu.VMEM((N, E), _f32),
            pltpu.VMEM((N, E), _f32),
            pltpu.VMEM((N, 1), _f32),
            pltpu.VMEM((N, 1), _f32),
        ],
    )(x, grad, wx, wgrow, rb)


@functools.cache
def _sc_mesh():
    return plsc.VectorSubcoreMesh(core_axis_name="c", subcore_axis_name="s",
                                  num_cores=NC, num_subcores=NS)


@functools.cache
def _get_dispatch():
    @functools.partial(
        pl.kernel,
        out_type=jax.ShapeDtypeStruct((P, D), _f32),
        mesh=_sc_mesh(),
        scratch_types=[
            pltpu.VMEM((CH,), _i32),
            pltpu.VMEM((CH,), _i32),
            pltpu.VMEM((CH, D), _f32),
            pltpu.SemaphoreType.DMA,
            pltpu.SemaphoreType.DMA,
        ],
    )
    def _dispatch(x_hbm, pos0_hbm, pos1_hbm, xg_hbm, idx0_v, idx1_v, rows_v,
                  sem0, sem1):
        wid = jax.lax.axis_index("s") * NC + jax.lax.axis_index("c")
        base = wid * TOK_W

        def chunk(i, _):
            b = base + i * CH
            pltpu.sync_copy(pos0_hbm.at[pl.ds(b, CH)], idx0_v)
            pltpu.sync_copy(pos1_hbm.at[pl.ds(b, CH)], idx1_v)
            pltpu.sync_copy(x_hbm.at[pl.ds(b, CH)], rows_v)
            c0 = pltpu.async_copy(rows_v, xg_hbm.at[idx0_v], sem0)
            c1 = pltpu.async_copy(rows_v, xg_hbm.at[idx1_v], sem1)
            c0.wait()
            c1.wait()
            return 0

        jax.lax.fori_loop(0, TOK_W // CH, chunk, 0)

    return _dispatch


def _start_chunk(c, e, w1_any, w2_any, stgA, stgB, sem):
    j = c // 2
    slot = j % 2

    @pl.when(c % 2 == 0)
    def _():
        pltpu.make_async_copy(
            w1_any.at[e, pl.ds(j * CW1, CW1), :],
            stgA.at[pl.ds(slot * CW1, CW1), :], sem).start()

    @pl.when(c % 2 == 1)
    def _():
        pltpu.make_async_copy(
            w2_any.at[e, pl.ds(j * CW2, CW2), :],
            stgB.at[pl.ds(slot * CW2, CW2), :], sem).start()


def _wait_chunk(c, e, w1_any, w2_any, stgA, stgB, sem):
    j = c // 2
    slot = j % 2

    @pl.when(c % 2 == 0)
    def _():
        pltpu.make_async_copy(
            w1_any.at[e, pl.ds(j * CW1, CW1), :],
            stgA.at[pl.ds(slot * CW1, CW1), :], sem).wait()

    @pl.when(c % 2 == 1)
    def _():
        pltpu.make_async_copy(
            w2_any.at[e, pl.ds(j * CW2, CW2), :],
            stgB.at[pl.ds(slot * CW2, CW2), :], sem).wait()


def _cast_chunk(c, dstpar, stgA, stgB, w1bf, w2bf):
    j = c // 2
    slot = j % 2

    @pl.when(c % 2 == 0)
    def _():
        w1bf[dstpar, pl.ds(j * CW1, CW1), :] = (
            stgA[pl.ds(slot * CW1, CW1), :].astype(jnp.bfloat16))

    @pl.when(c % 2 == 1)
    def _():
        w2bf[dstpar, pl.ds(j * CW2, CW2), :] = (
            stgB[pl.ds(slot * CW2, CW2), :].astype(jnp.bfloat16))


def _ffn_body(bexp_s, pfe_s, plo_s, phi_s, flo_s, fhi_s, par_s,
              xg_ref, w1_any, b1_ref, w2_any, b2_ref, y_ref,
              w1bf, w2bf, stgA, stgB, semP, semF):
    r = pl.program_id(0)
    par = par_s[r]
    tpar = 1 - par
    e_cur = bexp_s[r]
    e_nx = pfe_s[r]

    def fin(c, _):
        _start_chunk(c, e_cur, w1_any, w2_any, stgA, stgB, semF)
        _wait_chunk(c, e_cur, w1_any, w2_any, stgA, stgB, semF)
        _cast_chunk(c, par, stgA, stgB, w1bf, w2bf)
        return 0

    jax.lax.fori_loop(flo_s[r], fhi_s[r], fin, 0)

    def fire(c, _):
        _start_chunk(c, e_nx, w1_any, w2_any, stgA, stgB, semP)
        return 0

    jax.lax.fori_loop(plo_s[r], phi_s[r], fire, 0)

    xb = xg_ref[...].astype(jnp.bfloat16)
    h = jnp.dot(xb, w1bf[par], preferred_element_type=_f32) + b1_ref[0]
    g = 0.5 * h * (1.0 + jax.lax.erf(h * 0.7071067811865476))
    y = jnp.dot(g.astype(jnp.bfloat16), w2bf[par],
                preferred_element_type=_f32) + b2_ref[0]
    y_ref[...] = y

    def drain_wait(c, _):
        _wait_chunk(c, e_nx, w1_any, w2_any, stgA, stgB, semP)
        return 0

    jax.lax.fori_loop(plo_s[r], phi_s[r], drain_wait, 0)

    def drain_cast(c, _):
        _cast_chunk(c, tpar, stgA, stgB, w1bf, w2bf)
        return 0

    jax.lax.fori_loop(plo_s[r], phi_s[r], drain_cast, 0)


def _ffn(scalars, xg, w1, b1, w2, b2):
    grid_spec = pltpu.PrefetchScalarGridSpec(
        num_scalar_prefetch=7,
        grid=(NBLK,),
        in_specs=[
            pl.BlockSpec((B_R, D), lambda r, *s: (r, 0)),
            pl.BlockSpec(memory_space=pltpu.HBM),
            pl.BlockSpec((1, 1, H), lambda r, *s: (s[0][r], 0, 0)),
            pl.BlockSpec(memory_space=pltpu.HBM),
            pl.BlockSpec((1, 1, D), lambda r, *s: (s[0][r], 0, 0)),
        ],
        out_specs=pl.BlockSpec((B_R, D), lambda r, *s: (r, 0)),
        scratch_shapes=[
            pltpu.VMEM((2, D, H), jnp.bfloat16),
            pltpu.VMEM((2, H, D), jnp.bfloat16),
            pltpu.VMEM((2 * CW1, H), _f32),
            pltpu.VMEM((2 * CW2, D), _f32),
            pltpu.SemaphoreType.DMA,
            pltpu.SemaphoreType.DMA,
        ],
    )
    return pl.pallas_call(
        _ffn_body,
        grid_spec=grid_spec,
        out_shape=jax.ShapeDtypeStruct((P, D), _f32),
    )(*scalars, xg, w1, b1.reshape(E, 1, H), w2, b2.reshape(E, 1, D))


@functools.cache
def _get_combine():
    @functools.partial(
        pl.kernel,
        out_type=jax.ShapeDtypeStruct((N, D), _f32),
        mesh=_sc_mesh(),
        scratch_types=[
            pltpu.VMEM((CH2,), _i32),
            pltpu.VMEM((CH2,), _i32),
            pltpu.VMEM((CH2, 16), _f32),
            pltpu.VMEM((CH2, 16), _f32),
            pltpu.VMEM((CH2, D), _f32),
            pltpu.VMEM((CH2, D), _f32),
            pltpu.VMEM((CH2, D), _f32),
            pltpu.SemaphoreType.DMA,
            pltpu.SemaphoreType.DMA,
        ],
    )
    def _combine(y_hbm, pos0_hbm, pos1_hbm, w0_hbm, w1_hbm, out_hbm,
                 idx0_v, idx1_v, w0_v, w1_v, r0_v, r1_v, o_v, semA, semB):
        wid = jax.lax.axis_index("s") * NC + jax.lax.axis_index("c")
        base = wid * TOK_W

        def chunk(i, _):
            b = base + i * CH2
            pltpu.sync_copy(pos0_hbm.at[pl.ds(b, CH2)], idx0_v)
            pltpu.sync_copy(pos1_hbm.at[pl.ds(b, CH2)], idx1_v)
            pltpu.sync_copy(w0_hbm.at[pl.ds(b, CH2)], w0_v)
            pltpu.sync_copy(w1_hbm.at[pl.ds(b, CH2)], w1_v)
            cA = pltpu.async_copy(y_hbm.at[idx0_v], r0_v, semA)
            cB = pltpu.async_copy(y_hbm.at[idx1_v], r1_v, semB)
            cA.wait()
            cB.wait()

            def trow(t, _):
                s0 = w0_v[t, :]
                s1 = w1_v[t, :]

                def jbody(j, _):
                    sl = pl.ds(j * 16, 16)
                    o_v[t, sl] = s0 * r0_v[t, sl] + s1 * r1_v[t, sl]
                    return 0

                jax.lax.fori_loop(0, D // 16, jbody, 0)
                return 0

            jax.lax.fori_loop(0, CH2, trow, 0)
            pltpu.sync_copy(o_v, out_hbm.at[pl.ds(b, CH2)])
            return 0

        jax.lax.fori_loop(0, TOK_W // CH2, chunk, 0)

    return _combine


def kernel(x, grad, router_W, router_b, W1, b1, W2, b2):
    wx = router_W[:D]
    wgrow = router_W[D:]
    rb = router_b.reshape(1, E)

    (probs, w0, w1, pos0, pos1, bexp,
     pfe, plo, phi, flo, fhi, par) = _router(x, grad, wx, wgrow, rb)
    pos0f = pos0.reshape(N)
    pos1f = pos1.reshape(N)
    scalars = tuple(a.reshape(NBLK) for a in
                    (bexp, pfe, plo, phi, flo, fhi, par))

    xg = _get_dispatch()(x, pos0f, pos1f)
    y = _ffn(scalars, xg, W1, b1, W2, b2)

    out = _get_combine()(y, pos0f, pos1f, w0, w1)
    return out, probs


# speedup vs baseline: 1.0681x; 1.0681x over previous
import functools

import jax
import jax.numpy as jnp
from jax.experimental import pallas as pl
from jax.experimental.pallas import tpu as pltpu
from jax.experimental.pallas import tpu_sc as plsc

N = 4096
D = 1024
E = 8
K = 2
H = 4 * D

B_R = 256
P = N * K + E * B_R
NBLK = P // B_R

TILE = 128
NT = N // TILE

NC, NS = 2, 16
NW = NC * NS
TOK_W = N // NW
CH = 32
CH2 = 16

NCHK = 16
CAP = 4
CW1 = D // (NCHK // 2)
CW2 = H // (NCHK // 2)

_f32 = jnp.float32
_i32 = jnp.int32


def _router_body(x_ref, grad_ref, wx_ref, wg_ref, rb_ref,
                 probs_ref, w0_ref, w1_ref, pos0_ref, pos1_ref, bexp_ref,
                 pfe_ref, plo_ref, phi_ref, flo_ref, fhi_ref, par_ref,
                 oh0_s, oh1_s, part0_s, part1_s):
    x = x_ref[...]
    logits = jnp.dot(x, wx_ref[...], preferred_element_type=_f32)
    logits = logits + grad_ref[...] * wg_ref[...] + rb_ref[...]

    m = jnp.max(logits, axis=1, keepdims=True)
    ex = jnp.exp(logits - m)
    probs = ex / jnp.sum(ex, axis=1, keepdims=True)
    probs_ref[...] = probs

    iota_e = jax.lax.broadcasted_iota(_i32, (N, E), 1)
    v0 = jnp.max(probs, axis=1, keepdims=True)
    e0 = jnp.min(jnp.where(probs == v0, iota_e, E), axis=1, keepdims=True)
    oh0 = (iota_e == e0).astype(_f32)
    pm = jnp.where(iota_e == e0, -1.0, probs)
    v1 = jnp.max(pm, axis=1, keepdims=True)
    e1 = jnp.min(jnp.where(pm == v1, iota_e, E), axis=1, keepdims=True)
    oh1 = (iota_e == e1).astype(_f32)

    w0_ref[...] = jnp.broadcast_to(v0, (N, 16))
    w1_ref[...] = jnp.broadcast_to(v1, (N, 16))
    oh0_s[...] = oh0
    oh1_s[...] = oh1

    ti = jax.lax.broadcasted_iota(_i32, (TILE, TILE), 0)
    tj = jax.lax.broadcasted_iota(_i32, (TILE, TILE), 1)
    tri = (tj < ti).astype(_f32)

    def tile_body(b, carry):
        run0, run1 = carry
        o0 = oh0_s[pl.ds(b * TILE, TILE), :]
        o1 = oh1_s[pl.ds(b * TILE, TILE), :]
        ex0 = jnp.dot(tri, o0, preferred_element_type=_f32)
        ex1 = jnp.dot(tri, o1, preferred_element_type=_f32)
        part0_s[pl.ds(b * TILE, TILE), :] = jnp.sum(o0 * (ex0 + run0), axis=1,
                                                    keepdims=True)
        part1_s[pl.ds(b * TILE, TILE), :] = jnp.sum(o1 * (ex1 + run1), axis=1,
                                                    keepdims=True)
        return (run0 + jnp.sum(o0, axis=0, keepdims=True),
                run1 + jnp.sum(o1, axis=0, keepdims=True))

    run0, run1 = jax.lax.fori_loop(
        0, NT, tile_body,
        (jnp.zeros((1, E), _f32), jnp.zeros((1, E), _f32)))

    tot = (run0 + run1).astype(_i32)
    padded = ((tot + (B_R - 1)) // B_R) * B_R

    offs = [jnp.zeros((1, 1), _i32)]
    acc = jnp.zeros((1, 1), _i32)
    for j in range(1, E):
        acc = acc + padded[:, j - 1:j]
        offs.append(acc)
    offi = jnp.concatenate(offs, axis=1)
    off = offi.astype(_f32)
    ends = off + padded.astype(_f32)

    pos0 = part0_s[...] + jnp.sum(oh0 * off, axis=1, keepdims=True)
    pos1 = part1_s[...] + jnp.sum(oh1 * (off + run0), axis=1, keepdims=True)
    pos0_ref[...] = pos0.astype(_i32)
    pos1_ref[...] = pos1.astype(_i32)

    rstart = (jax.lax.broadcasted_iota(_i32, (NBLK, E), 0) * B_R).astype(_f32)
    cnt = jnp.sum((rstart >= ends).astype(_i32), axis=1, keepdims=True)
    bexp_v = jnp.minimum(cnt, E - 1)
    bexp_ref[...] = bexp_v

    present = (padded > 0).astype(_i32)
    rs = offi // B_R
    runlen = padded // B_R

    ne_l, lp_l, ord_l = [], [], []
    for e in range(E):
        nv = jnp.full((1, 1), E, _i32)
        for j in range(E - 1, e, -1):
            nv = jnp.where(present[:, j:j + 1] == 1, j, nv)
        ne_l.append(nv)
        pv = jnp.full((1, 1), -1, _i32)
        lp = jnp.zeros((1, 1), _i32)
        for j in range(0, e):
            pv = jnp.where(present[:, j:j + 1] == 1, j, pv)
        for j in range(0, e):
            lp = jnp.where(pv == j, runlen[:, j:j + 1], lp)
        lp_l.append(lp)
        ov = jnp.zeros((1, 1), _i32)
        for j in range(0, e):
            ov = ov + present[:, j:j + 1]
        ord_l.append(ov)
    ne = jnp.concatenate(ne_l, axis=1)
    lenprev = jnp.concatenate(lp_l, axis=1)
    ordv = jnp.concatenate(ord_l, axis=1)

    iota_be = jax.lax.broadcasted_iota(_i32, (NBLK, E), 1)
    ohb = (iota_be == bexp_v).astype(_i32)

    def sel(v):
        return jnp.sum(ohb * v, axis=1, keepdims=True)

    rs_b = sel(rs)
    pfe_b = sel(ne)
    lp_b = sel(lenprev)
    par_b = sel(ordv) % 2
    present_b = sel(present)
    rblk = jax.lax.broadcasted_iota(_i32, (NBLK, 1), 0)
    i_b = rblk - rs_b
    has_next = (pfe_b < E).astype(_i32)
    pfe_ref[...] = jnp.minimum(pfe_b, E - 1)
    plo_ref[...] = jnp.minimum(NCHK, i_b * CAP) * has_next
    phi_ref[...] = jnp.minimum(NCHK, (i_b + 1) * CAP) * has_next
    runstart = (i_b == 0).astype(_i32) * present_b
    done_prev = jnp.minimum(NCHK, lp_b * CAP)
    flo_ref[...] = jnp.where(runstart == 1, done_prev, 0)
    fhi_ref[...] = jnp.where(runstart == 1, NCHK, 0)
    par_ref[...] = par_b


def _router(x, grad, wx, wgrow, rb):
    return pl.pallas_call(
        _router_body,
        out_shape=(
            jax.ShapeDtypeStruct((N, E), _f32),
            jax.ShapeDtypeStruct((N, 16), _f32),
            jax.ShapeDtypeStruct((N, 16), _f32),
            jax.ShapeDtypeStruct((N, 1), _i32),
            jax.ShapeDtypeStruct((N, 1), _i32),
            jax.ShapeDtypeStruct((NBLK, 1), _i32),
            jax.ShapeDtypeStruct((NBLK, 1), _i32),
            jax.ShapeDtypeStruct((NBLK, 1), _i32),
            jax.ShapeDtypeStruct((NBLK, 1), _i32),
            jax.ShapeDtypeStruct((NBLK, 1), _i32),
            jax.ShapeDtypeStruct((NBLK, 1), _i32),
            jax.ShapeDtypeStruct((NBLK, 1), _i32),
        ),
        scratch_shapes=[
            pltpu.VMEM((N, E), _f32),
            pltpu.VMEM((N, E), _f32),
            pltpu.VMEM((N, 1), _f32),
            pltpu.VMEM((N, 1), _f32),
        ],
    )(x, grad, wx, wgrow, rb)


@functools.cache
def _sc_mesh():
    return plsc.VectorSubcoreMesh(core_axis_name="c", subcore_axis_name="s",
                                  num_cores=NC, num_subcores=NS)


@functools.cache
def _get_dispatch():
    @functools.partial(
        pl.kernel,
        out_type=jax.ShapeDtypeStruct((P, D), _f32),
        mesh=_sc_mesh(),
        scratch_types=[
            pltpu.VMEM((CH,), _i32),
            pltpu.VMEM((CH,), _i32),
            pltpu.VMEM((CH, D), _f32),
            pltpu.VMEM((CH, D), _f32),
            pltpu.SemaphoreType.DMA,
            pltpu.SemaphoreType.DMA,
            pltpu.SemaphoreType.DMA,
            pltpu.SemaphoreType.DMA,
        ],
    )
    def _dispatch(x_hbm, pos0_hbm, pos1_hbm, xg_hbm, idx0_v, idx1_v,
                  rows_a, rows_b, sem_ra, sem_rb, sem_sa, sem_sb):
        wid = jax.lax.axis_index("s") * NC + jax.lax.axis_index("c")
        base = wid * TOK_W

        def read(i, rows_v, sem):
            pltpu.async_copy(x_hbm.at[pl.ds(base + i * CH, CH)], rows_v, sem)

        def scat(i, rows_v, sem_r, sem_s):
            b = base + i * CH
            pltpu.make_async_copy(x_hbm.at[pl.ds(b, CH)], rows_v, sem_r).wait()
            pltpu.sync_copy(pos0_hbm.at[pl.ds(b, CH)], idx0_v)
            pltpu.async_copy(rows_v, xg_hbm.at[idx0_v], sem_s)
            pltpu.sync_copy(pos1_hbm.at[pl.ds(b, CH)], idx1_v)
            pltpu.async_copy(rows_v, xg_hbm.at[idx1_v], sem_s)

        def swait(rows_v, idx_v, sem_s):
            pltpu.make_async_copy(rows_v, xg_hbm.at[idx_v], sem_s).wait()
            pltpu.make_async_copy(rows_v, xg_hbm.at[idx_v], sem_s).wait()

        npair = TOK_W // CH // 2
        read(0, rows_a, sem_ra)

        def pair(p, _):
            i0 = 2 * p
            read(i0 + 1, rows_b, sem_rb)
            scat(i0, rows_a, sem_ra, sem_sa)
            swait(rows_a, idx0_v, sem_sa)

            @pl.when(p + 1 < npair)
            def _():
                read(i0 + 2, rows_a, sem_ra)

            scat(i0 + 1, rows_b, sem_rb, sem_sb)
            swait(rows_b, idx0_v, sem_sb)
            return 0

        jax.lax.fori_loop(0, npair, pair, 0)

    return _dispatch


def _start_chunk(c, e, w1_any, w2_any, stgA, stgB, sem):
    j = c // 2
    slot = j % 2

    @pl.when(c % 2 == 0)
    def _():
        pltpu.make_async_copy(
            w1_any.at[e, pl.ds(j * CW1, CW1), :],
            stgA.at[pl.ds(slot * CW1, CW1), :], sem).start()

    @pl.when(c % 2 == 1)
    def _():
        pltpu.make_async_copy(
            w2_any.at[e, pl.ds(j * CW2, CW2), :],
            stgB.at[pl.ds(slot * CW2, CW2), :], sem).start()


def _wait_chunk(c, e, w1_any, w2_any, stgA, stgB, sem):
    j = c // 2
    slot = j % 2

    @pl.when(c % 2 == 0)
    def _():
        pltpu.make_async_copy(
            w1_any.at[e, pl.ds(j * CW1, CW1), :],
            stgA.at[pl.ds(slot * CW1, CW1), :], sem).wait()

    @pl.when(c % 2 == 1)
    def _():
        pltpu.make_async_copy(
            w2_any.at[e, pl.ds(j * CW2, CW2), :],
            stgB.at[pl.ds(slot * CW2, CW2), :], sem).wait()


def _cast_chunk(c, dstpar, stgA, stgB, w1bf, w2bf):
    j = c // 2
    slot = j % 2

    @pl.when(c % 2 == 0)
    def _():
        w1bf[dstpar, pl.ds(j * CW1, CW1), :] = (
            stgA[pl.ds(slot * CW1, CW1), :].astype(jnp.bfloat16))

    @pl.when(c % 2 == 1)
    def _():
        w2bf[dstpar, pl.ds(j * CW2, CW2), :] = (
            stgB[pl.ds(slot * CW2, CW2), :].astype(jnp.bfloat16))


def _ffn_body(bexp_s, pfe_s, plo_s, phi_s, flo_s, fhi_s, par_s,
              xg_ref, w1_any, b1_ref, w2_any, b2_ref, y_ref,
              w1bf, w2bf, stgA, stgB, semP, semF):
    r = pl.program_id(0)
    par = par_s[r]
    tpar = 1 - par
    e_cur = bexp_s[r]
    e_nx = pfe_s[r]

    def fin(c, _):
        _start_chunk(c, e_cur, w1_any, w2_any, stgA, stgB, semF)
        _wait_chunk(c, e_cur, w1_any, w2_any, stgA, stgB, semF)
        _cast_chunk(c, par, stgA, stgB, w1bf, w2bf)
        return 0

    jax.lax.fori_loop(flo_s[r], fhi_s[r], fin, 0)

    def fire(c, _):
        _start_chunk(c, e_nx, w1_any, w2_any, stgA, stgB, semP)
        return 0

    jax.lax.fori_loop(plo_s[r], phi_s[r], fire, 0)

    xb = xg_ref[...].astype(jnp.bfloat16)
    h = jnp.dot(xb, w1bf[par], preferred_element_type=_f32) + b1_ref[0]
    g = 0.5 * h * (1.0 + jax.lax.erf(h * 0.7071067811865476))
    y = jnp.dot(g.astype(jnp.bfloat16), w2bf[par],
                preferred_element_type=_f32) + b2_ref[0]
    y_ref[...] = y

    def drain_wait(c, _):
        _wait_chunk(c, e_nx, w1_any, w2_any, stgA, stgB, semP)
        return 0

    jax.lax.fori_loop(plo_s[r], phi_s[r], drain_wait, 0)

    def drain_cast(c, _):
        _cast_chunk(c, tpar, stgA, stgB, w1bf, w2bf)
        return 0

    jax.lax.fori_loop(plo_s[r], phi_s[r], drain_cast, 0)


def _ffn(scalars, xg, w1, b1, w2, b2):
    grid_spec = pltpu.PrefetchScalarGridSpec(
        num_scalar_prefetch=7,
        grid=(NBLK,),
        in_specs=[
            pl.BlockSpec((B_R, D), lambda r, *s: (r, 0)),
            pl.BlockSpec(memory_space=pltpu.HBM),
            pl.BlockSpec((1, 1, H), lambda r, *s: (s[0][r], 0, 0)),
            pl.BlockSpec(memory_space=pltpu.HBM),
            pl.BlockSpec((1, 1, D), lambda r, *s: (s[0][r], 0, 0)),
        ],
        out_specs=pl.BlockSpec((B_R, D), lambda r, *s: (r, 0)),
        scratch_shapes=[
            pltpu.VMEM((2, D, H), jnp.bfloat16),
            pltpu.VMEM((2, H, D), jnp.bfloat16),
            pltpu.VMEM((2 * CW1, H), _f32),
            pltpu.VMEM((2 * CW2, D), _f32),
            pltpu.SemaphoreType.DMA,
            pltpu.SemaphoreType.DMA,
        ],
    )
    return pl.pallas_call(
        _ffn_body,
        grid_spec=grid_spec,
        out_shape=jax.ShapeDtypeStruct((P, D), _f32),
    )(*scalars, xg, w1, b1.reshape(E, 1, H), w2, b2.reshape(E, 1, D))


@functools.cache
def _get_combine():
    @functools.partial(
        pl.kernel,
        out_type=jax.ShapeDtypeStruct((N, D), _f32),
        mesh=_sc_mesh(),
        scratch_types=[
            pltpu.VMEM((CH2,), _i32),
            pltpu.VMEM((CH2,), _i32),
            pltpu.VMEM((CH2,), _i32),
            pltpu.VMEM((CH2,), _i32),
            pltpu.VMEM((CH2, 16), _f32),
            pltpu.VMEM((CH2, 16), _f32),
            pltpu.VMEM((CH2, D), _f32),
            pltpu.VMEM((CH2, D), _f32),
            pltpu.VMEM((CH2, D), _f32),
            pltpu.VMEM((CH2, D), _f32),
            pltpu.VMEM((CH2, D), _f32),
            pltpu.SemaphoreType.DMA,
            pltpu.SemaphoreType.DMA,
            pltpu.SemaphoreType.DMA,
            pltpu.SemaphoreType.DMA,
        ],
    )
    def _combine(y_hbm, pos0_hbm, pos1_hbm, w0_hbm, w1_hbm, out_hbm,
                 i0a, i1a, i0b, i1b, w0_v, w1_v,
                 r0a, r1a, r0b, r1b, o_v,
                 s0a, s1a, s0b, s1b):
        wid = jax.lax.axis_index("s") * NC + jax.lax.axis_index("c")
        base = wid * TOK_W

        def fire(ci, i0_v, i1_v, r0_v, r1_v, s0, s1):
            b = base + ci * CH2
            pltpu.sync_copy(pos0_hbm.at[pl.ds(b, CH2)], i0_v)
            pltpu.sync_copy(pos1_hbm.at[pl.ds(b, CH2)], i1_v)
            pltpu.async_copy(y_hbm.at[i0_v], r0_v, s0)
            pltpu.async_copy(y_hbm.at[i1_v], r1_v, s1)

        def finish(ci, i0_v, i1_v, r0_v, r1_v, s0, s1):
            b = base + ci * CH2
            pltpu.make_async_copy(y_hbm.at[i0_v], r0_v, s0).wait()
            pltpu.make_async_copy(y_hbm.at[i1_v], r1_v, s1).wait()
            pltpu.sync_copy(w0_hbm.at[pl.ds(b, CH2)], w0_v)
            pltpu.sync_copy(w1_hbm.at[pl.ds(b, CH2)], w1_v)

            def trow(t, _):
                s0v = w0_v[t, :]
                s1v = w1_v[t, :]
                for j in range(D // 16):
                    sl = slice(j * 16, (j + 1) * 16)
                    o_v[t, sl] = s0v * r0_v[t, sl] + s1v * r1_v[t, sl]
                return 0

            jax.lax.fori_loop(0, CH2, trow, 0)
            pltpu.sync_copy(o_v, out_hbm.at[pl.ds(b, CH2)])

        npair = TOK_W // CH2 // 2
        fire(0, i0a, i1a, r0a, r1a, s0a, s1a)

        def pair(p, _):
            i0 = 2 * p
            fire(i0 + 1, i0b, i1b, r0b, r1b, s0b, s1b)
            finish(i0, i0a, i1a, r0a, r1a, s0a, s1a)

            @pl.when(p + 1 < npair)
            def _():
                fire(i0 + 2, i0a, i1a, r0a, r1a, s0a, s1a)

            finish(i0 + 1, i0b, i1b, r0b, r1b, s0b, s1b)
            return 0

        jax.lax.fori_loop(0, npair, pair, 0)

    return _combine


def kernel(x, grad, router_W, router_b, W1, b1, W2, b2):
    wx = router_W[:D]
    wgrow = router_W[D:]
    rb = router_b.reshape(1, E)

    (probs, w0, w1, pos0, pos1, bexp,
     pfe, plo, phi, flo, fhi, par) = _router(x, grad, wx, wgrow, rb)
    pos0f = pos0.reshape(N)
    pos1f = pos1.reshape(N)
    scalars = tuple(a.reshape(NBLK) for a in
                    (bexp, pfe, plo, phi, flo, fhi, par))

    xg = _get_dispatch()(x, pos0f, pos1f)
    y = _ffn(scalars, xg, W1, b1, W2, b2)

    out = _get_combine()(y, pos0f, pos1f, w0, w1)
    return out, probs
